# baseline (speedup 1.0000x reference)
import jax
import jax.numpy as jnp
from jax.experimental import pallas as pl
from jax.experimental.pallas import tpu as pltpu

_N = 100000
_D = 512
_H = 128
_K = 8
_TILE = 2000
_GRID = _N // _TILE


def _dot_t(a, b):
    return jax.lax.dot_general(a, b, (((1,), (1,)), ((), ())),
                               preferred_element_type=jnp.float32)


def _stream_kernel(feat_a_ref, feat_b_ref, W_fc_ref, b_fc_ref, W_a1_ref,
                   b_a1_ref, W_a2_ref, b_a2_ref, W_ic_ref, b_ic_ref,
                   a_ref, l4_ref, m_ref, z_ref, s_ref):
    h = (_dot_t(feat_a_ref[...], W_fc_ref[:, :_D // 2]) +
         _dot_t(feat_b_ref[...], W_fc_ref[:, _D // 2:]) + b_fc_ref[...])
    t = jnp.tanh(_dot_t(h, W_a1_ref[...]) + b_a1_ref[...])
    a_row = _dot_t(W_a2_ref[...], t) + b_a2_ref[...]
    l4_ref[...] = (_dot_t(W_ic_ref[...], h) + b_ic_ref[...]).reshape(1, 4, 8, _TILE // 8)
    a_ref[...] = a_row.reshape(1, 8, _TILE // 8)
    m_i = jnp.max(a_row)
    w = jnp.exp(a_row - m_i)
    m_ref[...] = jnp.full((1, 1, 1), m_i)
    z_ref[...] = jnp.sum(w).reshape(1, 1, 1)
    s_ref[...] = jax.lax.dot_general(
        w, h, (((1,), (0,)), ((), ())),
        preferred_element_type=jnp.float32).reshape(1, 1, _H)


def _finalize_kernel(a_ref, l4_ref, m_ref, z_ref, s_ref, label_ref,
                     out_m_ref, out_loss_ref):
    m = m_ref[...]
    gm = jnp.max(m)
    scale = jnp.exp(m - gm)
    Z = jnp.sum(z_ref[...] * scale)
    out_m_ref[...] = jnp.sum(s_ref[...] * scale, axis=0) / Z

    a2 = a_ref[...].reshape(_GRID * 8, _TILE // 8)
    lane = jax.lax.broadcasted_iota(jnp.int32, (1, _TILE // 8), 1)

    def pick_cols(cv, largest):
        fill = -jnp.inf if largest else jnp.inf
        rows = []
        v = cv
        for _ in range(_K):
            best = jnp.max(v) if largest else jnp.min(v)
            idx = jnp.min(jnp.where(v == best, lane, jnp.int32(2 ** 30)))
            oh = lane == idx
            rows.append(oh.astype(jnp.float32))
            v = jnp.where(oh, fill, v)
        return rows

    colmax = jnp.max(a2, axis=0, keepdims=True)
    colmin = jnp.min(a2, axis=0, keepdims=True)
    sel = jnp.concatenate(pick_cols(colmax, True) + pick_cols(colmin, False),
                          axis=0)

    def _compact(x):
        return jax.lax.dot_general(x, sel, (((1,), (1,)), ((), ())),
                                   precision=jax.lax.Precision.HIGHEST,
                                   preferred_element_type=jnp.float32)

    ca = _compact(a2)
    cl = [_compact(l4_ref[:, j, :, :].reshape(_GRID * 8, _TILE // 8))
          for j in range(4)]
    ciota = jax.lax.broadcasted_iota(jnp.int32, ca.shape, 0) * 16 + \
        jax.lax.broadcasted_iota(jnp.int32, ca.shape, 1)

    def winner_mask(cols, largest):
        fill = -jnp.inf if largest else jnp.inf
        v = ca[:, cols]
        ic = ciota[:, cols]
        wmask = jnp.zeros_like(v, dtype=jnp.bool_)
        for _ in range(_K):
            best = jnp.max(v) if largest else jnp.min(v)
            eq = v == best
            first = jnp.min(jnp.where(eq, ic, jnp.int32(2 ** 30)))
            oh = jnp.logical_and(eq, ic == first)
            wmask = jnp.logical_or(wmask, oh)
            v = jnp.where(oh, fill, v)
        return wmask

    wtop = winner_mask(slice(0, _K), largest=True)
    wbot = winner_mask(slice(_K, 2 * _K), largest=False)

    lab = label_ref[...]
    total = jnp.zeros((1, 1), jnp.float32)
    for cls in range(2):
        l0, l1 = cl[2 * cls], cl[2 * cls + 1]
        mx = jnp.maximum(l0, l1)
        lse = mx + jnp.log(jnp.exp(l0 - mx) + jnp.exp(l1 - mx))
        acc = (jnp.sum(jnp.where(wtop, (lse - l1)[:, :_K], 0.0)) +
               jnp.sum(jnp.where(wbot, (lse - l0)[:, _K:], 0.0)))
        inst = jnp.reshape(acc / (2 * _K), (1, 1))
        total = total + jnp.where(lab[0:1, cls:cls + 1] == 1, inst, 0.0)
    out_loss_ref[...] = total


def kernel(features, label, W_fc, b_fc, W_a1, b_a1, W_a2, b_a2, W_ic, b_ic):
    W_ic4 = W_ic.reshape(2 * 2, _H)
    b_ic4 = b_ic.reshape(2 * 2, 1)
    full2 = lambda shape: pl.BlockSpec(shape, lambda i: (0, 0))
    a, l4, m, z, s = pl.pallas_call(
        _stream_kernel,
        grid=(_GRID,),
        in_specs=[
            pl.BlockSpec((_TILE, _D // 2), lambda i: (i, 0)),
            pl.BlockSpec((_TILE, _D // 2), lambda i: (i, 1)),
            full2((_H, _D)),
            full2((1, _H)),
            full2((_H, _H)),
            full2((1, _H)),
            full2((1, _H)),
            full2((1, 1)),
            full2((4, _H)),
            full2((4, 1)),
        ],
        out_specs=[
            pl.BlockSpec((1, 8, _TILE // 8), lambda i: (i, 0, 0)),
            pl.BlockSpec((1, 4, 8, _TILE // 8), lambda i: (i, 0, 0, 0)),
            pl.BlockSpec((1, 1, 1), lambda i: (i, 0, 0)),
            pl.BlockSpec((1, 1, 1), lambda i: (i, 0, 0)),
            pl.BlockSpec((1, 1, _H), lambda i: (i, 0, 0)),
        ],
        out_shape=[
            jax.ShapeDtypeStruct((_GRID, 8, _TILE // 8), jnp.float32),
            jax.ShapeDtypeStruct((_GRID, 4, 8, _TILE // 8), jnp.float32),
            jax.ShapeDtypeStruct((_GRID, 1, 1), jnp.float32),
            jax.ShapeDtypeStruct((_GRID, 1, 1), jnp.float32),
            jax.ShapeDtypeStruct((_GRID, 1, _H), jnp.float32),
        ],
        compiler_params=pltpu.CompilerParams(
            dimension_semantics=("parallel",)),
    )(features, features, W_fc, b_fc.reshape(1, _H), W_a1,
      b_a1.reshape(1, _H), W_a2, b_a2.reshape(1, 1), W_ic4, b_ic4)

    M, loss = pl.pallas_call(
        _finalize_kernel,
        out_shape=[
            jax.ShapeDtypeStruct((1, _H), jnp.float32),
            jax.ShapeDtypeStruct((1, 1), jnp.float32),
        ],
    )(a, l4, m, z, s, label.reshape(1, 2))
    return (M, loss[0, 0])


# speedup vs baseline: 1.2212x; 1.2212x over previous
import jax
import jax.numpy as jnp
from jax.experimental import pallas as pl
from jax.experimental.pallas import tpu as pltpu

_N = 100000
_D = 512
_H = 128
_K = 8
_TILE = 4000
_GRID = _N // _TILE


def _dot_t(a, b):
    return jax.lax.dot_general(a, b, (((1,), (1,)), ((), ())),
                               preferred_element_type=jnp.float32)


def _stream_kernel(feat_a_ref, feat_b_ref, W_fc_ref, b_fc_ref, W_a1_ref,
                   b_a1_ref, W_a2_ref, b_a2_ref, W_ic_ref, b_ic_ref,
                   a_ref, l4_ref, m_ref, z_ref, s_ref):
    h = (_dot_t(feat_a_ref[...], W_fc_ref[:, :_D // 2]) +
         _dot_t(feat_b_ref[...], W_fc_ref[:, _D // 2:]) + b_fc_ref[...])
    t = jnp.tanh(_dot_t(h, W_a1_ref[...]) + b_a1_ref[...])
    a_row = _dot_t(W_a2_ref[...], t) + b_a2_ref[...]
    l4_ref[...] = (_dot_t(W_ic_ref[...], h) + b_ic_ref[...]).reshape(1, 4, 8, _TILE // 8)
    a_ref[...] = a_row.reshape(1, 8, _TILE // 8)
    m_i = jnp.max(a_row)
    w = jnp.exp(a_row - m_i)
    m_ref[...] = jnp.full((1, 1, 1), m_i)
    z_ref[...] = jnp.sum(w).reshape(1, 1, 1)
    s_ref[...] = jax.lax.dot_general(
        w, h, (((1,), (0,)), ((), ())),
        preferred_element_type=jnp.float32).reshape(1, 1, _H)


def _finalize_kernel(a_ref, l4_ref, m_ref, z_ref, s_ref, label_ref,
                     out_m_ref, out_loss_ref):
    m = m_ref[...]
    gm = jnp.max(m)
    scale = jnp.exp(m - gm)
    Z = jnp.sum(z_ref[...] * scale)
    out_m_ref[...] = jnp.sum(s_ref[...] * scale, axis=0) / Z

    a2 = a_ref[...].reshape(_GRID * 8, _TILE // 8)
    lane = jax.lax.broadcasted_iota(jnp.int32, (1, _TILE // 8), 1)

    def pick_cols(cv, largest):
        fill = -jnp.inf if largest else jnp.inf
        rows = []
        v = cv
        for _ in range(_K):
            best = jnp.max(v) if largest else jnp.min(v)
            idx = jnp.min(jnp.where(v == best, lane, jnp.int32(2 ** 30)))
            oh = lane == idx
            rows.append(oh.astype(jnp.float32))
            v = jnp.where(oh, fill, v)
        return rows

    colmax = jnp.max(a2, axis=0, keepdims=True)
    colmin = jnp.min(a2, axis=0, keepdims=True)
    sel = jnp.concatenate(pick_cols(colmax, True) + pick_cols(colmin, False),
                          axis=0)

    def _compact(x):
        return jax.lax.dot_general(x, sel, (((1,), (1,)), ((), ())),
                                   precision=jax.lax.Precision.HIGHEST,
                                   preferred_element_type=jnp.float32)

    ca = _compact(a2)
    cl = [_compact(l4_ref[:, j, :, :].reshape(_GRID * 8, _TILE // 8))
          for j in range(4)]
    ciota = jax.lax.broadcasted_iota(jnp.int32, ca.shape, 0) * 16 + \
        jax.lax.broadcasted_iota(jnp.int32, ca.shape, 1)

    def winner_mask(cols, largest):
        fill = -jnp.inf if largest else jnp.inf
        v = ca[:, cols]
        ic = ciota[:, cols]
        wmask = jnp.zeros_like(v, dtype=jnp.bool_)
        for _ in range(_K):
            best = jnp.max(v) if largest else jnp.min(v)
            eq = v == best
            first = jnp.min(jnp.where(eq, ic, jnp.int32(2 ** 30)))
            oh = jnp.logical_and(eq, ic == first)
            wmask = jnp.logical_or(wmask, oh)
            v = jnp.where(oh, fill, v)
        return wmask

    wtop = winner_mask(slice(0, _K), largest=True)
    wbot = winner_mask(slice(_K, 2 * _K), largest=False)

    lab = label_ref[...]
    total = jnp.zeros((1, 1), jnp.float32)
    for cls in range(2):
        l0, l1 = cl[2 * cls], cl[2 * cls + 1]
        mx = jnp.maximum(l0, l1)
        lse = mx + jnp.log(jnp.exp(l0 - mx) + jnp.exp(l1 - mx))
        acc = (jnp.sum(jnp.where(wtop, (lse - l1)[:, :_K], 0.0)) +
               jnp.sum(jnp.where(wbot, (lse - l0)[:, _K:], 0.0)))
        inst = jnp.reshape(acc / (2 * _K), (1, 1))
        total = total + jnp.where(lab[0:1, cls:cls + 1] == 1, inst, 0.0)
    out_loss_ref[...] = total


def kernel(features, label, W_fc, b_fc, W_a1, b_a1, W_a2, b_a2, W_ic, b_ic):
    W_ic4 = W_ic.reshape(2 * 2, _H)
    b_ic4 = b_ic.reshape(2 * 2, 1)
    full2 = lambda shape: pl.BlockSpec(shape, lambda i: (0, 0))
    a, l4, m, z, s = pl.pallas_call(
        _stream_kernel,
        grid=(_GRID,),
        in_specs=[
            pl.BlockSpec((_TILE, _D // 2), lambda i: (i, 0)),
            pl.BlockSpec((_TILE, _D // 2), lambda i: (i, 1)),
            full2((_H, _D)),
            full2((1, _H)),
            full2((_H, _H)),
            full2((1, _H)),
            full2((1, _H)),
            full2((1, 1)),
            full2((4, _H)),
            full2((4, 1)),
        ],
        out_specs=[
            pl.BlockSpec((1, 8, _TILE // 8), lambda i: (i, 0, 0)),
            pl.BlockSpec((1, 4, 8, _TILE // 8), lambda i: (i, 0, 0, 0)),
            pl.BlockSpec((1, 1, 1), lambda i: (i, 0, 0)),
            pl.BlockSpec((1, 1, 1), lambda i: (i, 0, 0)),
            pl.BlockSpec((1, 1, _H), lambda i: (i, 0, 0)),
        ],
        out_shape=[
            jax.ShapeDtypeStruct((_GRID, 8, _TILE // 8), jnp.float32),
            jax.ShapeDtypeStruct((_GRID, 4, 8, _TILE // 8), jnp.float32),
            jax.ShapeDtypeStruct((_GRID, 1, 1), jnp.float32),
            jax.ShapeDtypeStruct((_GRID, 1, 1), jnp.float32),
            jax.ShapeDtypeStruct((_GRID, 1, _H), jnp.float32),
        ],
        compiler_params=pltpu.CompilerParams(
            dimension_semantics=("parallel",)),
    )(features, features, W_fc, b_fc.reshape(1, _H), W_a1,
      b_a1.reshape(1, _H), W_a2, b_a2.reshape(1, 1), W_ic4, b_ic4)

    M, loss = pl.pallas_call(
        _finalize_kernel,
        out_shape=[
            jax.ShapeDtypeStruct((1, _H), jnp.float32),
            jax.ShapeDtypeStruct((1, 1), jnp.float32),
        ],
    )(a, l4, m, z, s, label.reshape(1, 2))
    return (M, loss[0, 0])


# speedup vs baseline: 1.3233x; 1.0836x over previous
import jax
import jax.numpy as jnp
from jax.experimental import pallas as pl
from jax.experimental.pallas import tpu as pltpu

_N = 100000
_D = 512
_H = 128
_K = 8
_TILE = 10000
_GRID = _N // _TILE


def _dot_t(a, b):
    return jax.lax.dot_general(a, b, (((1,), (1,)), ((), ())),
                               preferred_element_type=jnp.float32)


def _stream_kernel(feat_a_ref, feat_b_ref, W_fc_ref, b_fc_ref, W_a1_ref,
                   b_a1_ref, W_a2_ref, b_a2_ref, W_ic_ref, b_ic_ref,
                   a_ref, l4_ref, m_ref, z_ref, s_ref):
    h = (_dot_t(feat_a_ref[...], W_fc_ref[:, :_D // 2]) +
         _dot_t(feat_b_ref[...], W_fc_ref[:, _D // 2:]) + b_fc_ref[...])
    t = jnp.tanh(_dot_t(h, W_a1_ref[...]) + b_a1_ref[...])
    a_row = _dot_t(W_a2_ref[...], t) + b_a2_ref[...]
    l4_ref[...] = (_dot_t(W_ic_ref[...], h) + b_ic_ref[...]).reshape(1, 4, 8, _TILE // 8)
    a_ref[...] = a_row.reshape(1, 8, _TILE // 8)
    m_i = jnp.max(a_row)
    w = jnp.exp(a_row - m_i)
    m_ref[...] = jnp.full((1, 1, 1), m_i)
    z_ref[...] = jnp.sum(w).reshape(1, 1, 1)
    s_ref[...] = jax.lax.dot_general(
        w, h, (((1,), (0,)), ((), ())),
        preferred_element_type=jnp.float32).reshape(1, 1, _H)


def _finalize_kernel(a_ref, l4_ref, m_ref, z_ref, s_ref, label_ref,
                     out_m_ref, out_loss_ref):
    m = m_ref[...]
    gm = jnp.max(m)
    scale = jnp.exp(m - gm)
    Z = jnp.sum(z_ref[...] * scale)
    out_m_ref[...] = jnp.sum(s_ref[...] * scale, axis=0) / Z

    a2 = a_ref[...].reshape(_GRID * 8, _TILE // 8)
    lane = jax.lax.broadcasted_iota(jnp.int32, (1, _TILE // 8), 1)

    def pick_cols(cv, largest):
        fill = -jnp.inf if largest else jnp.inf
        rows = []
        v = cv
        for _ in range(_K):
            best = jnp.max(v) if largest else jnp.min(v)
            idx = jnp.min(jnp.where(v == best, lane, jnp.int32(2 ** 30)))
            oh = lane == idx
            rows.append(oh.astype(jnp.float32))
            v = jnp.where(oh, fill, v)
        return rows

    colmax = jnp.max(a2, axis=0, keepdims=True)
    colmin = jnp.min(a2, axis=0, keepdims=True)
    sel = jnp.concatenate(pick_cols(colmax, True) + pick_cols(colmin, False),
                          axis=0)

    def _compact(x):
        return jax.lax.dot_general(x, sel, (((1,), (1,)), ((), ())),
                                   precision=jax.lax.Precision.HIGHEST,
                                   preferred_element_type=jnp.float32)

    ca = _compact(a2)
    cl = [_compact(l4_ref[:, j, :, :].reshape(_GRID * 8, _TILE // 8))
          for j in range(4)]
    ciota = jax.lax.broadcasted_iota(jnp.int32, ca.shape, 0) * 16 + \
        jax.lax.broadcasted_iota(jnp.int32, ca.shape, 1)

    def winner_mask(cols, largest):
        fill = -jnp.inf if largest else jnp.inf
        v = ca[:, cols]
        ic = ciota[:, cols]
        wmask = jnp.zeros_like(v, dtype=jnp.bool_)
        for _ in range(_K):
            best = jnp.max(v) if largest else jnp.min(v)
            eq = v == best
            first = jnp.min(jnp.where(eq, ic, jnp.int32(2 ** 30)))
            oh = jnp.logical_and(eq, ic == first)
            wmask = jnp.logical_or(wmask, oh)
            v = jnp.where(oh, fill, v)
        return wmask

    wtop = winner_mask(slice(0, _K), largest=True)
    wbot = winner_mask(slice(_K, 2 * _K), largest=False)

    lab = label_ref[...]
    total = jnp.zeros((1, 1), jnp.float32)
    for cls in range(2):
        l0, l1 = cl[2 * cls], cl[2 * cls + 1]
        mx = jnp.maximum(l0, l1)
        lse = mx + jnp.log(jnp.exp(l0 - mx) + jnp.exp(l1 - mx))
        acc = (jnp.sum(jnp.where(wtop, (lse - l1)[:, :_K], 0.0)) +
               jnp.sum(jnp.where(wbot, (lse - l0)[:, _K:], 0.0)))
        inst = jnp.reshape(acc / (2 * _K), (1, 1))
        total = total + jnp.where(lab[0:1, cls:cls + 1] == 1, inst, 0.0)
    out_loss_ref[...] = total


def kernel(features, label, W_fc, b_fc, W_a1, b_a1, W_a2, b_a2, W_ic, b_ic):
    W_ic4 = W_ic.reshape(2 * 2, _H)
    b_ic4 = b_ic.reshape(2 * 2, 1)
    full2 = lambda shape: pl.BlockSpec(shape, lambda i: (0, 0))
    a, l4, m, z, s = pl.pallas_call(
        _stream_kernel,
        grid=(_GRID,),
        in_specs=[
            pl.BlockSpec((_TILE, _D // 2), lambda i: (i, 0)),
            pl.BlockSpec((_TILE, _D // 2), lambda i: (i, 1)),
            full2((_H, _D)),
            full2((1, _H)),
            full2((_H, _H)),
            full2((1, _H)),
            full2((1, _H)),
            full2((1, 1)),
            full2((4, _H)),
            full2((4, 1)),
        ],
        out_specs=[
            pl.BlockSpec((1, 8, _TILE // 8), lambda i: (i, 0, 0)),
            pl.BlockSpec((1, 4, 8, _TILE // 8), lambda i: (i, 0, 0, 0)),
            pl.BlockSpec((1, 1, 1), lambda i: (i, 0, 0)),
            pl.BlockSpec((1, 1, 1), lambda i: (i, 0, 0)),
            pl.BlockSpec((1, 1, _H), lambda i: (i, 0, 0)),
        ],
        out_shape=[
            jax.ShapeDtypeStruct((_GRID, 8, _TILE // 8), jnp.float32),
            jax.ShapeDtypeStruct((_GRID, 4, 8, _TILE // 8), jnp.float32),
            jax.ShapeDtypeStruct((_GRID, 1, 1), jnp.float32),
            jax.ShapeDtypeStruct((_GRID, 1, 1), jnp.float32),
            jax.ShapeDtypeStruct((_GRID, 1, _H), jnp.float32),
        ],
        compiler_params=pltpu.CompilerParams(
            dimension_semantics=("parallel",)),
    )(features, features, W_fc, b_fc.reshape(1, _H), W_a1,
      b_a1.reshape(1, _H), W_a2, b_a2.reshape(1, 1), W_ic4, b_ic4)

    M, loss = pl.pallas_call(
        _finalize_kernel,
        out_shape=[
            jax.ShapeDtypeStruct((1, _H), jnp.float32),
            jax.ShapeDtypeStruct((1, 1), jnp.float32),
        ],
    )(a, l4, m, z, s, label.reshape(1, 2))
    return (M, loss[0, 0])
